# baseline (speedup 1.0000x reference)
import functools

import jax
import jax.numpy as jnp
from jax.experimental import pallas as pl


def _split(x):
    hi = x.astype(jnp.bfloat16)
    r = x - hi.astype(jnp.float32)
    mid = r.astype(jnp.bfloat16)
    lo = (r - mid.astype(jnp.float32)).astype(jnp.bfloat16)
    return hi, mid, lo


def _body(mb, na, nd, nf, nbf, hid, ncls,
          atoms_ref, bonds_ref, edges_ref,
          w1a_ref, b1_ref, w2a_ref, b2_ref,
          wball_ref, woa_ref, bo_ref,
          wf_ref, bf_ref, out_ref):
    ndeg = nd + 1
    atoms = atoms_ref[...]
    edges = edges_ref[...]
    bonds = bonds_ref[...]

    valid = edges != -1
    deg = jnp.sum(valid.astype(jnp.int32), axis=2)
    deg_col = deg.reshape(mb * na, 1)
    nz_col = (deg_col > 0).astype(jnp.float32)
    dlane = jax.lax.broadcasted_iota(jnp.int32, (mb * na, ndeg * hid), 1) // hid
    mask_cat = (dlane == deg_col).astype(jnp.float32)

    bsum = bonds[:, :, 0:nbf]
    for d in range(1, nd):
        bsum = bsum + bonds[:, :, d * nbf:(d + 1) * nbf]
    b6 = bsum.reshape(mb * na, nbf)
    bb_all = jnp.dot(b6, wball_ref[...],
                     preferred_element_type=jnp.float32)
    bb1 = bb_all[:, 0:ndeg * hid]
    bb2 = bb_all[:, ndeg * hid:2 * ndeg * hid]
    bbo = bb_all[:, 2 * ndeg * hid:]

    col = jax.lax.broadcasted_iota(jnp.int32, (mb, na, na), 2)
    onehots = []
    for d in range(nd):
        e_d = edges[:, :, d]
        onehots.append((e_d[:, :, None] == col).astype(jnp.bfloat16))
    adj = onehots[0]
    for d in range(1, nd):
        adj = adj + onehots[d]
    oh_cat = jnp.concatenate(onehots, axis=1)

    def bmm(a, b):
        return jax.lax.dot_general(
            a, b, (((2,), (1,)), ((0,), (0,))),
            preferred_element_type=jnp.float32)

    def gather2(oh, x):
        hi, mid, lo = _split(x)
        return bmm(oh, hi) + bmm(oh, mid) + bmm(oh, lo)

    def conv(x, wa_ref, b_ref, bb, in_dim):
        summed = x + gather2(adj, x)
        s = summed.reshape(mb * na, in_dim)
        y = (jnp.dot(s, wa_ref[...], preferred_element_type=jnp.float32)
             + bb + b_ref[...]) * mask_cat
        out = y[:, 0:hid]
        for d in range(1, ndeg):
            out = out + y[:, d * hid:(d + 1) * hid]
        return jnp.maximum(out, 0.0).reshape(mb, na, hid)

    def pool(x):
        g_all = gather2(oh_cat, x)
        p = x
        for d in range(nd):
            p = jnp.maximum(p, g_all[:, d * na:(d + 1) * na, :])
        return p * nz_col.reshape(mb, na, 1)

    x1 = conv(atoms, w1a_ref, b1_ref, bb1, nf)
    p1 = pool(x1)
    x2 = conv(p1, w2a_ref, b2_ref, bb2, hid)
    p2 = pool(x2)

    f = jnp.tanh(jnp.dot(p2.reshape(mb * na, hid), woa_ref[...],
                         preferred_element_type=jnp.float32)
                 + bbo + bo_ref[...])
    f = f * nz_col
    fp = f.reshape(mb, na, hid).sum(axis=1)
    logits = (jnp.dot(fp, wf_ref[...], preferred_element_type=jnp.float32)
              + bf_ref[...])
    out_ref[...] = jax.nn.sigmoid(logits)


@jax.jit
def kernel(atoms, bonds, edges, W1, b1, W2, b2, Wo, bo, Wf, bf):
    bm, na, nf = atoms.shape
    nd = edges.shape[-1]
    nbf = bonds.shape[-1]
    ndeg, in1, hid = W1.shape
    ncls = Wf.shape[-1]
    mb = 8
    grid = (bm // mb,)

    bonds_r = bonds.reshape(bm, na, nd * nbf)
    w1c = W1.transpose(1, 0, 2).reshape(in1, ndeg * hid)
    w1a, w1b = w1c[:nf], w1c[nf:]
    b1c = b1.reshape(1, ndeg * hid)
    in2 = W2.shape[1]
    w2c = W2.transpose(1, 0, 2).reshape(in2, ndeg * hid)
    w2a, w2b = w2c[:hid], w2c[hid:]
    b2c = b2.reshape(1, ndeg * hid)
    woa, wob = Wo[:hid], Wo[hid:]
    wball = jnp.concatenate([w1b, w2b, wob], axis=1)
    bo2 = bo.reshape(1, hid)
    bf2 = bf.reshape(1, ncls)

    const = lambda *shape: pl.BlockSpec(shape, lambda i: (0,) * len(shape))
    return pl.pallas_call(
        functools.partial(_body, mb, na, nd, nf, nbf, hid, ncls),
        grid=grid,
        in_specs=[
            pl.BlockSpec((mb, na, nf), lambda i: (i, 0, 0)),
            pl.BlockSpec((mb, na, nd * nbf), lambda i: (i, 0, 0)),
            pl.BlockSpec((mb, na, nd), lambda i: (i, 0, 0)),
            const(nf, ndeg * hid),
            const(1, ndeg * hid),
            const(hid, ndeg * hid),
            const(1, ndeg * hid),
            const(nbf, 2 * ndeg * hid + hid),
            const(hid, hid),
            const(1, hid),
            const(hid, ncls),
            const(1, ncls),
        ],
        out_specs=pl.BlockSpec((mb, ncls), lambda i: (i, 0)),
        out_shape=jax.ShapeDtypeStruct((bm, ncls), jnp.float32),
    )(atoms, bonds_r, edges, w1a, b1c, w2a, b2c,
      wball, woa, bo2, Wf, bf2)


# speedup vs baseline: 1.0494x; 1.0494x over previous
import functools

import jax
import jax.numpy as jnp
from jax.experimental import pallas as pl


def _split(x):
    hi = x.astype(jnp.bfloat16)
    r = x - hi.astype(jnp.float32)
    mid = r.astype(jnp.bfloat16)
    lo = (r - mid.astype(jnp.float32)).astype(jnp.bfloat16)
    return hi, mid, lo


def _body(mb, na, nd, nf, nbf, hid, ncls,
          atoms_ref, bonds_ref, edges_ref,
          w1a_ref, b1_ref, w2a_ref, b2_ref,
          wball_ref, woa_ref, bo_ref,
          wf_ref, bf_ref, out_ref):
    ndeg = nd + 1
    atoms = atoms_ref[...]
    edges = edges_ref[...]
    bonds = bonds_ref[...]

    valid = edges != -1
    deg = jnp.sum(valid.astype(jnp.int32), axis=2)
    deg_col = deg.reshape(mb * na, 1)
    nz_col = (deg_col > 0).astype(jnp.float32)
    dlane = jax.lax.broadcasted_iota(jnp.int32, (mb * na, ndeg * hid), 1) // hid
    mask_cat = (dlane == deg_col).astype(jnp.float32)

    bsum = bonds[:, :, 0:nbf]
    for d in range(1, nd):
        bsum = bsum + bonds[:, :, d * nbf:(d + 1) * nbf]
    b6 = bsum.reshape(mb * na, nbf)
    bb_all = jnp.dot(b6, wball_ref[...],
                     preferred_element_type=jnp.float32)
    bb1 = bb_all[:, 0:ndeg * hid]
    bb2 = bb_all[:, ndeg * hid:2 * ndeg * hid]
    bbo = bb_all[:, 2 * ndeg * hid:]

    col = jax.lax.broadcasted_iota(jnp.int32, (mb, na, na), 2)
    onehots = []
    for d in range(nd):
        e_d = edges[:, :, d]
        onehots.append((e_d[:, :, None] == col).astype(jnp.bfloat16))
    adj = onehots[0]
    for d in range(1, nd):
        adj = adj + onehots[d]
    oh_cat = jnp.concatenate(onehots, axis=1)

    def bmm(a, b):
        return jax.lax.dot_general(
            a, b, (((2,), (1,)), ((0,), (0,))),
            preferred_element_type=jnp.float32)

    def gather2(oh, x):
        hi, mid, lo = _split(x)
        return bmm(oh, hi) + bmm(oh, mid) + bmm(oh, lo)

    def conv(x, wa_ref, b_ref, bb, in_dim):
        summed = x + gather2(adj, x)
        s = summed.reshape(mb * na, in_dim)
        y = (jnp.dot(s, wa_ref[...], preferred_element_type=jnp.float32)
             + bb + b_ref[...]) * mask_cat
        out = y[:, 0:hid]
        for d in range(1, ndeg):
            out = out + y[:, d * hid:(d + 1) * hid]
        return jnp.maximum(out, 0.0).reshape(mb, na, hid)

    def pool(x):
        g_all = gather2(oh_cat, x)
        p = x
        for d in range(nd):
            p = jnp.maximum(p, g_all[:, d * na:(d + 1) * na, :])
        return p * nz_col.reshape(mb, na, 1)

    x1 = conv(atoms, w1a_ref, b1_ref, bb1, nf)
    p1 = pool(x1)
    x2 = conv(p1, w2a_ref, b2_ref, bb2, hid)
    p2 = pool(x2)

    f = jnp.tanh(jnp.dot(p2.reshape(mb * na, hid), woa_ref[...],
                         preferred_element_type=jnp.float32)
                 + bbo + bo_ref[...])
    f = f * nz_col
    fp = f.reshape(mb, na, hid).sum(axis=1)
    logits = (jnp.dot(fp, wf_ref[...], preferred_element_type=jnp.float32)
              + bf_ref[...])
    out_ref[...] = jax.nn.sigmoid(logits)


@jax.jit
def kernel(atoms, bonds, edges, W1, b1, W2, b2, Wo, bo, Wf, bf):
    bm, na, nf = atoms.shape
    nd = edges.shape[-1]
    nbf = bonds.shape[-1]
    ndeg, in1, hid = W1.shape
    ncls = Wf.shape[-1]
    mb = 16
    grid = (bm // mb,)

    bonds_r = bonds.reshape(bm, na, nd * nbf)
    w1c = W1.transpose(1, 0, 2).reshape(in1, ndeg * hid)
    w1a, w1b = w1c[:nf], w1c[nf:]
    b1c = b1.reshape(1, ndeg * hid)
    in2 = W2.shape[1]
    w2c = W2.transpose(1, 0, 2).reshape(in2, ndeg * hid)
    w2a, w2b = w2c[:hid], w2c[hid:]
    b2c = b2.reshape(1, ndeg * hid)
    woa, wob = Wo[:hid], Wo[hid:]
    wball = jnp.concatenate([w1b, w2b, wob], axis=1)
    bo2 = bo.reshape(1, hid)
    bf2 = bf.reshape(1, ncls)

    const = lambda *shape: pl.BlockSpec(shape, lambda i: (0,) * len(shape))
    return pl.pallas_call(
        functools.partial(_body, mb, na, nd, nf, nbf, hid, ncls),
        grid=grid,
        in_specs=[
            pl.BlockSpec((mb, na, nf), lambda i: (i, 0, 0)),
            pl.BlockSpec((mb, na, nd * nbf), lambda i: (i, 0, 0)),
            pl.BlockSpec((mb, na, nd), lambda i: (i, 0, 0)),
            const(nf, ndeg * hid),
            const(1, ndeg * hid),
            const(hid, ndeg * hid),
            const(1, ndeg * hid),
            const(nbf, 2 * ndeg * hid + hid),
            const(hid, hid),
            const(1, hid),
            const(hid, ncls),
            const(1, ncls),
        ],
        out_specs=pl.BlockSpec((mb, ncls), lambda i: (i, 0)),
        out_shape=jax.ShapeDtypeStruct((bm, ncls), jnp.float32),
    )(atoms, bonds_r, edges, w1a, b1c, w2a, b2c,
      wball, woa, bo2, Wf, bf2)


# speedup vs baseline: 1.0582x; 1.0084x over previous
import functools

import jax
import jax.numpy as jnp
from jax.experimental import pallas as pl


def _split(x):
    hi = x.astype(jnp.bfloat16)
    r = x - hi.astype(jnp.float32)
    mid = r.astype(jnp.bfloat16)
    lo = (r - mid.astype(jnp.float32)).astype(jnp.bfloat16)
    return hi, mid, lo


def _body(mb, na, nd, nf, nbf, hid, ncls,
          atoms_ref, bonds_ref, edges_ref,
          w1a_ref, w2a_ref, woa_ref,
          t6_ref, wrb_ref, wrbias_ref,
          wf_ref, bf_ref, out_ref):
    ndeg = nd + 1
    atoms = atoms_ref[...]
    edges = edges_ref[...]
    bonds = bonds_ref[...]

    valid = edges != -1
    deg = jnp.sum(valid.astype(jnp.int32), axis=2)
    deg_col = deg.reshape(mb * na, 1)
    nz_col = (deg_col > 0).astype(jnp.float32)
    dlane = jax.lax.broadcasted_iota(jnp.int32, (mb * na, ndeg * hid), 1) // hid
    mask_cat = (dlane == deg_col).astype(jnp.float32)

    bsum = bonds[:, :, 0:nbf]
    for d in range(1, nd):
        bsum = bsum + bonds[:, :, d * nbf:(d + 1) * nbf]
    b6 = bsum.reshape(mb * na, nbf)
    kbp = 48
    b6t = jnp.dot(b6, t6_ref[...],
                  preferred_element_type=jnp.float32)
    l48 = jax.lax.broadcasted_iota(jnp.int32, (mb * na, kbp), 1) // nbf
    krp = b6t * (l48 == deg_col).astype(jnp.float32)
    l8 = jax.lax.broadcasted_iota(jnp.int32, (mb * na, 8), 1)
    dsel = (l8 == deg_col).astype(jnp.float32)
    q = (jnp.dot(krp, wrb_ref[...], preferred_element_type=jnp.float32)
         + jnp.dot(dsel, wrbias_ref[...],
                   preferred_element_type=jnp.float32))
    q1 = q[:, 0:hid]
    q2 = q[:, hid:2 * hid]
    qo = q[:, 2 * hid:]

    col = jax.lax.broadcasted_iota(jnp.int32, (mb, na, na), 2)
    onehots = []
    for d in range(nd):
        e_d = edges[:, :, d]
        onehots.append((e_d[:, :, None] == col).astype(jnp.bfloat16))
    adj = onehots[0]
    for d in range(1, nd):
        adj = adj + onehots[d]
    oh_cat = jnp.concatenate(onehots, axis=1)

    def bmm(a, b):
        return jax.lax.dot_general(
            a, b, (((2,), (1,)), ((0,), (0,))),
            preferred_element_type=jnp.float32)

    def gather2(oh, x):
        hi, mid, lo = _split(x)
        return bmm(oh, hi) + bmm(oh, mid) + bmm(oh, lo)

    def conv(x, wa_ref, qq, in_dim):
        summed = x + gather2(adj, x)
        s = summed.reshape(mb * na, in_dim)
        y = jnp.dot(s, wa_ref[...],
                    preferred_element_type=jnp.float32) * mask_cat
        out = qq + y[:, 0:hid]
        for d in range(1, ndeg):
            out = out + y[:, d * hid:(d + 1) * hid]
        return jnp.maximum(out, 0.0).reshape(mb, na, hid)

    def pool(x):
        g_all = gather2(oh_cat, x)
        p = x
        for d in range(nd):
            p = jnp.maximum(p, g_all[:, d * na:(d + 1) * na, :])
        return p * nz_col.reshape(mb, na, 1)

    x1 = conv(atoms, w1a_ref, q1, nf)
    p1 = pool(x1)
    x2 = conv(p1, w2a_ref, q2, hid)
    p2 = pool(x2)

    f = jnp.tanh(jnp.dot(p2.reshape(mb * na, hid), woa_ref[...],
                         preferred_element_type=jnp.float32)
                 + qo)
    f = f * nz_col
    fp = f.reshape(mb, na, hid).sum(axis=1)
    logits = (jnp.dot(fp, wf_ref[...], preferred_element_type=jnp.float32)
              + bf_ref[...])
    out_ref[...] = jax.nn.sigmoid(logits)


@jax.jit
def kernel(atoms, bonds, edges, W1, b1, W2, b2, Wo, bo, Wf, bf):
    bm, na, nf = atoms.shape
    nd = edges.shape[-1]
    nbf = bonds.shape[-1]
    ndeg, in1, hid = W1.shape
    ncls = Wf.shape[-1]
    mb = 16
    grid = (bm // mb,)

    bonds_r = bonds.reshape(bm, na, nd * nbf)
    w1c = W1.transpose(1, 0, 2).reshape(in1, ndeg * hid)
    w1a, w1b = w1c[:nf], w1c[nf:]
    b1c = b1.reshape(1, ndeg * hid)
    in2 = W2.shape[1]
    w2c = W2.transpose(1, 0, 2).reshape(in2, ndeg * hid)
    w2a, w2b = w2c[:hid], w2c[hid:]
    b2c = b2.reshape(1, ndeg * hid)
    woa, wob = Wo[:hid], Wo[hid:]
    bf2 = bf.reshape(1, ncls)
    kb, kbp = ndeg * nbf, 48
    t6 = jnp.concatenate(
        [jnp.eye(nbf, dtype=jnp.float32)] * ndeg
        + [jnp.zeros((nbf, kbp - kb), jnp.float32)], axis=1)
    w1b_r = w1b.reshape(nbf, ndeg, hid).transpose(1, 0, 2).reshape(kb, hid)
    w2b_r = w2b.reshape(nbf, ndeg, hid).transpose(1, 0, 2).reshape(kb, hid)
    wob_r = jnp.tile(wob, (ndeg, 1))
    wrb = jnp.concatenate(
        [jnp.concatenate([w1b_r, w2b_r, wob_r], axis=1),
         jnp.zeros((kbp - kb, 3 * hid), jnp.float32)], axis=0)
    wrbias = jnp.concatenate(
        [jnp.concatenate([b1, b2, jnp.tile(bo[None], (ndeg, 1))], axis=1),
         jnp.zeros((1, 3 * hid), jnp.float32)], axis=0)

    const = lambda *shape: pl.BlockSpec(shape, lambda i: (0,) * len(shape))
    return pl.pallas_call(
        functools.partial(_body, mb, na, nd, nf, nbf, hid, ncls),
        grid=grid,
        in_specs=[
            pl.BlockSpec((mb, na, nf), lambda i: (i, 0, 0)),
            pl.BlockSpec((mb, na, nd * nbf), lambda i: (i, 0, 0)),
            pl.BlockSpec((mb, na, nd), lambda i: (i, 0, 0)),
            const(nf, ndeg * hid),
            const(hid, ndeg * hid),
            const(hid, hid),
            const(nbf, 48),
            const(48, 3 * hid),
            const(8, 3 * hid),
            const(hid, ncls),
            const(1, ncls),
        ],
        out_specs=pl.BlockSpec((mb, ncls), lambda i: (i, 0)),
        out_shape=jax.ShapeDtypeStruct((bm, ncls), jnp.float32),
    )(atoms, bonds_r, edges, w1a, w2a, woa,
      t6, wrb, wrbias, Wf, bf2)


# speedup vs baseline: 1.0628x; 1.0043x over previous
import functools

import jax
import jax.numpy as jnp
from jax.experimental import pallas as pl


def _split(x):
    hi = x.astype(jnp.bfloat16)
    r = x - hi.astype(jnp.float32)
    mid = r.astype(jnp.bfloat16)
    lo = (r - mid.astype(jnp.float32)).astype(jnp.bfloat16)
    return hi, mid, lo


def _body(mb, na, nd, nf, nbf, hid, ncls,
          atoms_ref, bonds_ref, edges_ref,
          w1a_ref, w2a_ref, woa_ref,
          t6_ref, wrb_ref, wrbias_ref,
          wf_ref, bf_ref, out_ref):
    ndeg = nd + 1
    atoms = atoms_ref[...]
    edges = edges_ref[...]
    bonds = bonds_ref[...]

    valid = edges != -1
    deg = jnp.sum(valid.astype(jnp.int32), axis=2)
    deg_col = deg.reshape(mb * na, 1)
    nz_col = (deg_col > 0).astype(jnp.float32)
    dlane = jax.lax.broadcasted_iota(jnp.int32, (mb * na, ndeg * hid), 1) // hid
    mask_cat = (dlane == deg_col).astype(jnp.float32)

    bsum = bonds[:, :, 0:nbf]
    for d in range(1, nd):
        bsum = bsum + bonds[:, :, d * nbf:(d + 1) * nbf]
    b6 = bsum.reshape(mb * na, nbf)
    kbp = 48
    b6t = jnp.dot(b6, t6_ref[...],
                  preferred_element_type=jnp.float32)
    l48 = jax.lax.broadcasted_iota(jnp.int32, (mb * na, kbp), 1) // nbf
    krp = b6t * (l48 == deg_col).astype(jnp.float32)
    l8 = jax.lax.broadcasted_iota(jnp.int32, (mb * na, 8), 1)
    dsel = (l8 == deg_col).astype(jnp.float32)
    q = (jnp.dot(krp, wrb_ref[...], preferred_element_type=jnp.float32)
         + jnp.dot(dsel, wrbias_ref[...],
                   preferred_element_type=jnp.float32))
    q1 = q[:, 0:hid]
    q2 = q[:, hid:2 * hid]
    qo = q[:, 2 * hid:]

    col = jax.lax.broadcasted_iota(jnp.int32, (mb, na, na), 2)
    onehots = []
    for d in range(nd):
        e_d = edges[:, :, d]
        onehots.append((e_d[:, :, None] == col).astype(jnp.bfloat16))
    adj = onehots[0]
    for d in range(1, nd):
        adj = adj + onehots[d]
    oh_cat = jnp.concatenate(onehots, axis=1)

    def bmm(a, b):
        return jax.lax.dot_general(
            a, b, (((2,), (1,)), ((0,), (0,))),
            preferred_element_type=jnp.float32)

    def gather2(oh, x):
        hi, mid, lo = _split(x)
        return bmm(oh, hi) + bmm(oh, mid) + bmm(oh, lo)

    def conv(x, wa_ref, qq, in_dim):
        summed = x + gather2(adj, x)
        s = summed.reshape(mb * na, in_dim)
        y = jnp.dot(s.astype(jnp.bfloat16), wa_ref[...],
                    preferred_element_type=jnp.float32) * mask_cat
        out = qq + y[:, 0:hid]
        for d in range(1, ndeg):
            out = out + y[:, d * hid:(d + 1) * hid]
        return jnp.maximum(out, 0.0).reshape(mb, na, hid)

    def pool(x):
        g_all = gather2(oh_cat, x)
        p = x
        for d in range(nd):
            p = jnp.maximum(p, g_all[:, d * na:(d + 1) * na, :])
        return p * nz_col.reshape(mb, na, 1)

    x1 = conv(atoms, w1a_ref, q1, nf)
    p1 = pool(x1)
    x2 = conv(p1, w2a_ref, q2, hid)
    p2 = pool(x2)

    f = jnp.tanh(jnp.dot(p2.reshape(mb * na, hid).astype(jnp.bfloat16),
                         woa_ref[...],
                         preferred_element_type=jnp.float32)
                 + qo)
    f = f * nz_col
    fp = f.reshape(mb, na, hid).sum(axis=1)
    logits = (jnp.dot(fp.astype(jnp.bfloat16), wf_ref[...],
                      preferred_element_type=jnp.float32)
              + bf_ref[...])
    out_ref[...] = jax.nn.sigmoid(logits)


@jax.jit
def kernel(atoms, bonds, edges, W1, b1, W2, b2, Wo, bo, Wf, bf):
    bm, na, nf = atoms.shape
    nd = edges.shape[-1]
    nbf = bonds.shape[-1]
    ndeg, in1, hid = W1.shape
    ncls = Wf.shape[-1]
    mb = 16
    grid = (bm // mb,)

    bonds_r = bonds.reshape(bm, na, nd * nbf)
    w1c = W1.transpose(1, 0, 2).reshape(in1, ndeg * hid)
    w1a, w1b = w1c[:nf], w1c[nf:]
    b1c = b1.reshape(1, ndeg * hid)
    in2 = W2.shape[1]
    w2c = W2.transpose(1, 0, 2).reshape(in2, ndeg * hid)
    w2a, w2b = w2c[:hid], w2c[hid:]
    b2c = b2.reshape(1, ndeg * hid)
    woa, wob = Wo[:hid], Wo[hid:]
    bf2 = bf.reshape(1, ncls)
    kb, kbp = ndeg * nbf, 48
    t6 = jnp.concatenate(
        [jnp.eye(nbf, dtype=jnp.float32)] * ndeg
        + [jnp.zeros((nbf, kbp - kb), jnp.float32)], axis=1)
    w1b_r = w1b.reshape(nbf, ndeg, hid).transpose(1, 0, 2).reshape(kb, hid)
    w2b_r = w2b.reshape(nbf, ndeg, hid).transpose(1, 0, 2).reshape(kb, hid)
    wob_r = jnp.tile(wob, (ndeg, 1))
    wrb = jnp.concatenate(
        [jnp.concatenate([w1b_r, w2b_r, wob_r], axis=1),
         jnp.zeros((kbp - kb, 3 * hid), jnp.float32)], axis=0)
    wrbias = jnp.concatenate(
        [jnp.concatenate([b1, b2, jnp.tile(bo[None], (ndeg, 1))], axis=1),
         jnp.zeros((1, 3 * hid), jnp.float32)], axis=0)

    const = lambda *shape: pl.BlockSpec(shape, lambda i: (0,) * len(shape))
    return pl.pallas_call(
        functools.partial(_body, mb, na, nd, nf, nbf, hid, ncls),
        grid=grid,
        in_specs=[
            pl.BlockSpec((mb, na, nf), lambda i: (i, 0, 0)),
            pl.BlockSpec((mb, na, nd * nbf), lambda i: (i, 0, 0)),
            pl.BlockSpec((mb, na, nd), lambda i: (i, 0, 0)),
            const(nf, ndeg * hid),
            const(hid, ndeg * hid),
            const(hid, hid),
            const(nbf, 48),
            const(48, 3 * hid),
            const(8, 3 * hid),
            const(hid, ncls),
            const(1, ncls),
        ],
        out_specs=pl.BlockSpec((mb, ncls), lambda i: (i, 0)),
        out_shape=jax.ShapeDtypeStruct((bm, ncls), jnp.float32),
    )(atoms, bonds_r, edges,
      w1a.astype(jnp.bfloat16), w2a.astype(jnp.bfloat16),
      woa.astype(jnp.bfloat16), t6, wrb, wrbias,
      Wf.astype(jnp.bfloat16), bf2)


# speedup vs baseline: 1.1012x; 1.0361x over previous
import functools

import jax
import jax.numpy as jnp
from jax.experimental import pallas as pl


def _split(x):
    hi = x.astype(jnp.bfloat16)
    r = x - hi.astype(jnp.float32)
    mid = r.astype(jnp.bfloat16)
    lo = (r - mid.astype(jnp.float32)).astype(jnp.bfloat16)
    return hi, mid, lo


def _body(mb, na, nd, nf, nbf, hid, ncls,
          atoms_ref, bonds_ref, edges_ref,
          w1a_ref, w2a_ref, woa_ref,
          t6_ref, wrb_ref, wrbias_ref,
          wf_ref, bf_ref, out_ref):
    ndeg = nd + 1
    atoms = atoms_ref[...]
    edges = edges_ref[...]
    bonds = bonds_ref[...]

    valid = edges != -1
    deg = jnp.sum(valid.astype(jnp.int32), axis=2)
    deg_col = deg.reshape(mb * na, 1)
    nz_col = (deg_col > 0).astype(jnp.float32)
    dlane = jax.lax.broadcasted_iota(jnp.int32, (mb * na, ndeg * hid), 1) // hid
    mask_cat = (dlane == deg_col).astype(jnp.float32)

    bsum = bonds[:, :, 0:nbf]
    for d in range(1, nd):
        bsum = bsum + bonds[:, :, d * nbf:(d + 1) * nbf]
    b6 = bsum.reshape(mb * na, nbf)
    kbp = 48
    b6t = jnp.dot(b6, t6_ref[...],
                  preferred_element_type=jnp.float32)
    l48 = jax.lax.broadcasted_iota(jnp.int32, (mb * na, kbp), 1) // nbf
    krp = b6t * (l48 == deg_col).astype(jnp.float32)
    l8 = jax.lax.broadcasted_iota(jnp.int32, (mb * na, 8), 1)
    dsel = (l8 == deg_col).astype(jnp.float32)
    q = (jnp.dot(krp, wrb_ref[...], preferred_element_type=jnp.float32)
         + jnp.dot(dsel, wrbias_ref[...],
                   preferred_element_type=jnp.float32))
    q1 = q[:, 0:hid]
    q2 = q[:, hid:2 * hid]
    qo = q[:, 2 * hid:]

    col = jax.lax.broadcasted_iota(jnp.int32, (mb, na, na), 2)
    onehots = []
    for d in range(nd):
        e_d = edges[:, :, d]
        onehots.append((e_d[:, :, None] == col).astype(jnp.bfloat16))
    adj = onehots[0]
    for d in range(1, nd):
        adj = adj + onehots[d]
    oh_cat = jnp.concatenate(onehots, axis=1)

    def bmm(a, b):
        return jax.lax.dot_general(
            a, b, (((2,), (1,)), ((0,), (0,))),
            preferred_element_type=jnp.float32)

    def gather2(oh, x):
        hi, mid, lo = _split(x)
        return bmm(oh, hi) + bmm(oh, mid) + bmm(oh, lo)

    def conv(x, wa_ref, qq, in_dim):
        summed = x + gather2(adj, x)
        s = summed.reshape(mb * na, in_dim)
        y = jnp.dot(s.astype(jnp.bfloat16), wa_ref[...],
                    preferred_element_type=jnp.float32) * mask_cat
        out = qq + y[:, 0:hid]
        for d in range(1, ndeg):
            out = out + y[:, d * hid:(d + 1) * hid]
        return jnp.maximum(out, 0.0).reshape(mb, na, hid)

    def pool(x):
        g_all = gather2(oh_cat, x)
        p = x
        for d in range(nd):
            p = jnp.maximum(p, g_all[:, d * na:(d + 1) * na, :])
        return p * nz_col.reshape(mb, na, 1)

    x1 = conv(atoms, w1a_ref, q1, nf)
    p1 = pool(x1)
    x2 = conv(p1, w2a_ref, q2, hid)
    p2 = pool(x2)

    f = jnp.tanh(jnp.dot(p2.reshape(mb * na, hid).astype(jnp.bfloat16),
                         woa_ref[...],
                         preferred_element_type=jnp.float32)
                 + qo)
    f = f * nz_col
    fp = f.reshape(mb, na, hid).sum(axis=1)
    logits = (jnp.dot(fp.astype(jnp.bfloat16), wf_ref[...],
                      preferred_element_type=jnp.float32)
              + bf_ref[...])
    out_ref[...] = jax.nn.sigmoid(logits)


@jax.jit
def kernel(atoms, bonds, edges, W1, b1, W2, b2, Wo, bo, Wf, bf):
    bm, na, nf = atoms.shape
    nd = edges.shape[-1]
    nbf = bonds.shape[-1]
    ndeg, in1, hid = W1.shape
    ncls = Wf.shape[-1]
    mb = 32
    grid = (bm // mb,)

    bonds_r = bonds.reshape(bm, na, nd * nbf)
    w1c = W1.transpose(1, 0, 2).reshape(in1, ndeg * hid)
    w1a, w1b = w1c[:nf], w1c[nf:]
    b1c = b1.reshape(1, ndeg * hid)
    in2 = W2.shape[1]
    w2c = W2.transpose(1, 0, 2).reshape(in2, ndeg * hid)
    w2a, w2b = w2c[:hid], w2c[hid:]
    b2c = b2.reshape(1, ndeg * hid)
    woa, wob = Wo[:hid], Wo[hid:]
    bf2 = bf.reshape(1, ncls)
    kb, kbp = ndeg * nbf, 48
    t6 = jnp.concatenate(
        [jnp.eye(nbf, dtype=jnp.float32)] * ndeg
        + [jnp.zeros((nbf, kbp - kb), jnp.float32)], axis=1)
    w1b_r = w1b.reshape(nbf, ndeg, hid).transpose(1, 0, 2).reshape(kb, hid)
    w2b_r = w2b.reshape(nbf, ndeg, hid).transpose(1, 0, 2).reshape(kb, hid)
    wob_r = jnp.tile(wob, (ndeg, 1))
    wrb = jnp.concatenate(
        [jnp.concatenate([w1b_r, w2b_r, wob_r], axis=1),
         jnp.zeros((kbp - kb, 3 * hid), jnp.float32)], axis=0)
    wrbias = jnp.concatenate(
        [jnp.concatenate([b1, b2, jnp.tile(bo[None], (ndeg, 1))], axis=1),
         jnp.zeros((1, 3 * hid), jnp.float32)], axis=0)

    const = lambda *shape: pl.BlockSpec(shape, lambda i: (0,) * len(shape))
    return pl.pallas_call(
        functools.partial(_body, mb, na, nd, nf, nbf, hid, ncls),
        grid=grid,
        in_specs=[
            pl.BlockSpec((mb, na, nf), lambda i: (i, 0, 0)),
            pl.BlockSpec((mb, na, nd * nbf), lambda i: (i, 0, 0)),
            pl.BlockSpec((mb, na, nd), lambda i: (i, 0, 0)),
            const(nf, ndeg * hid),
            const(hid, ndeg * hid),
            const(hid, hid),
            const(nbf, 48),
            const(48, 3 * hid),
            const(8, 3 * hid),
            const(hid, ncls),
            const(1, ncls),
        ],
        out_specs=pl.BlockSpec((mb, ncls), lambda i: (i, 0)),
        out_shape=jax.ShapeDtypeStruct((bm, ncls), jnp.float32),
    )(atoms, bonds_r, edges,
      w1a.astype(jnp.bfloat16), w2a.astype(jnp.bfloat16),
      woa.astype(jnp.bfloat16), t6, wrb, wrbias,
      Wf.astype(jnp.bfloat16), bf2)


# speedup vs baseline: 1.1989x; 1.0887x over previous
import functools

import jax
import jax.numpy as jnp
from jax.experimental import pallas as pl


def _split(x):
    hi = x.astype(jnp.bfloat16)
    r = x - hi.astype(jnp.float32)
    mid = r.astype(jnp.bfloat16)
    lo = (r - mid.astype(jnp.float32)).astype(jnp.bfloat16)
    return hi, mid, lo


def _body(mb, na, nd, nf, nbf, hid, ncls,
          atoms_ref, bonds_ref, edges_ref,
          w1a_ref, w2a_ref, woa_ref,
          st6_ref, ones8_ref, l8_ref, dl48_ref, dl896_ref,
          wrb_ref, wrbias_ref,
          wf_ref, bf_ref, out_ref):
    ndeg = nd + 1
    atoms = atoms_ref[...]
    edges = edges_ref[...]
    bonds = bonds_ref[...]

    vf = (edges != -1).astype(jnp.float32).reshape(mb * na, nd)
    deg8 = jnp.dot(vf, ones8_ref[...],
                   preferred_element_type=jnp.float32)
    deg_col = deg8[:, 0:1]
    nz_col = (deg_col > 0.0).astype(jnp.float32)
    mask_cat = (dl896_ref[...] == deg_col).astype(jnp.float32)

    bflat = bonds.reshape(mb * na, nd * nbf)
    bh = bflat.astype(jnp.bfloat16)
    bl = (bflat - bh.astype(jnp.float32)).astype(jnp.bfloat16)
    b6t = (jnp.dot(bh, st6_ref[...], preferred_element_type=jnp.float32)
           + jnp.dot(bl, st6_ref[...],
                     preferred_element_type=jnp.float32))
    krp = b6t * (dl48_ref[...] == deg_col).astype(jnp.float32)
    dsel = (l8_ref[...] == deg8).astype(jnp.float32)
    q = (jnp.dot(krp, wrb_ref[...], preferred_element_type=jnp.float32)
         + jnp.dot(dsel, wrbias_ref[...],
                   preferred_element_type=jnp.float32))
    q1 = q[:, 0:hid]
    q2 = q[:, hid:2 * hid]
    qo = q[:, 2 * hid:]

    col = jax.lax.broadcasted_iota(jnp.int32, (mb, na, na), 2)
    onehots = []
    for d in range(nd):
        e_d = edges[:, :, d]
        onehots.append((e_d[:, :, None] == col).astype(jnp.bfloat16))
    adj = onehots[0]
    for d in range(1, nd):
        adj = adj + onehots[d]
    oh_cat = jnp.concatenate(onehots, axis=1)

    def bmm(a, b):
        return jax.lax.dot_general(
            a, b, (((2,), (1,)), ((0,), (0,))),
            preferred_element_type=jnp.float32)

    def gather2(oh, x):
        hi, mid, lo = _split(x)
        return bmm(oh, hi) + bmm(oh, mid) + bmm(oh, lo)

    def conv(x, wa_ref, qq, in_dim):
        summed = x + gather2(adj, x)
        s = summed.reshape(mb * na, in_dim)
        y = jnp.dot(s.astype(jnp.bfloat16), wa_ref[...],
                    preferred_element_type=jnp.float32) * mask_cat
        out = qq + y[:, 0:hid]
        for d in range(1, ndeg):
            out = out + y[:, d * hid:(d + 1) * hid]
        return jnp.maximum(out, 0.0).reshape(mb, na, hid)

    def pool(x):
        g_all = gather2(oh_cat, x)
        p = x
        for d in range(nd):
            p = jnp.maximum(p, g_all[:, d * na:(d + 1) * na, :])
        return p * nz_col.reshape(mb, na, 1)

    x1 = conv(atoms, w1a_ref, q1, nf)
    p1 = pool(x1)
    x2 = conv(p1, w2a_ref, q2, hid)
    p2 = pool(x2)

    f = jnp.tanh(jnp.dot(p2.reshape(mb * na, hid).astype(jnp.bfloat16),
                         woa_ref[...],
                         preferred_element_type=jnp.float32)
                 + qo)
    f = f * nz_col
    fp = f.reshape(mb, na, hid).sum(axis=1)
    logits = (jnp.dot(fp.astype(jnp.bfloat16), wf_ref[...],
                      preferred_element_type=jnp.float32)
              + bf_ref[...])
    out_ref[...] = jax.nn.sigmoid(logits)


@jax.jit
def kernel(atoms, bonds, edges, W1, b1, W2, b2, Wo, bo, Wf, bf):
    bm, na, nf = atoms.shape
    nd = edges.shape[-1]
    nbf = bonds.shape[-1]
    ndeg, in1, hid = W1.shape
    ncls = Wf.shape[-1]
    mb = 32
    grid = (bm // mb,)

    bonds_r = bonds.reshape(bm, na, nd * nbf)
    w1c = W1.transpose(1, 0, 2).reshape(in1, ndeg * hid)
    w1a, w1b = w1c[:nf], w1c[nf:]
    b1c = b1.reshape(1, ndeg * hid)
    in2 = W2.shape[1]
    w2c = W2.transpose(1, 0, 2).reshape(in2, ndeg * hid)
    w2a, w2b = w2c[:hid], w2c[hid:]
    b2c = b2.reshape(1, ndeg * hid)
    woa, wob = Wo[:hid], Wo[hid:]
    bf2 = bf.reshape(1, ncls)
    kb, kbp = ndeg * nbf, 48
    st6 = jnp.tile(jnp.eye(nbf, dtype=jnp.float32), (nd, 8))[:, :kbp]
    ones8 = jnp.ones((nd, 8), jnp.float32)
    l8 = jnp.arange(8, dtype=jnp.float32).reshape(1, 8)
    dl48 = (jnp.arange(kbp, dtype=jnp.int32) // nbf).astype(jnp.float32).reshape(1, kbp)
    dl896 = (jnp.arange(ndeg * hid, dtype=jnp.int32) // hid).astype(
        jnp.float32).reshape(1, ndeg * hid)
    w1b_r = w1b.reshape(nbf, ndeg, hid).transpose(1, 0, 2).reshape(kb, hid)
    w2b_r = w2b.reshape(nbf, ndeg, hid).transpose(1, 0, 2).reshape(kb, hid)
    wob_r = jnp.tile(wob, (ndeg, 1))
    wrb = jnp.concatenate(
        [jnp.concatenate([w1b_r, w2b_r, wob_r], axis=1),
         jnp.zeros((kbp - kb, 3 * hid), jnp.float32)], axis=0)
    wrbias = jnp.concatenate(
        [jnp.concatenate([b1, b2, jnp.tile(bo[None], (ndeg, 1))], axis=1),
         jnp.zeros((1, 3 * hid), jnp.float32)], axis=0)

    const = lambda *shape: pl.BlockSpec(shape, lambda i: (0,) * len(shape))
    return pl.pallas_call(
        functools.partial(_body, mb, na, nd, nf, nbf, hid, ncls),
        grid=grid,
        in_specs=[
            pl.BlockSpec((mb, na, nf), lambda i: (i, 0, 0)),
            pl.BlockSpec((mb, na, nd * nbf), lambda i: (i, 0, 0)),
            pl.BlockSpec((mb, na, nd), lambda i: (i, 0, 0)),
            const(nf, ndeg * hid),
            const(hid, ndeg * hid),
            const(hid, hid),
            const(nd * nbf, 48),
            const(nd, 8),
            const(1, 8),
            const(1, 48),
            const(1, ndeg * hid),
            const(48, 3 * hid),
            const(8, 3 * hid),
            const(hid, ncls),
            const(1, ncls),
        ],
        out_specs=pl.BlockSpec((mb, ncls), lambda i: (i, 0)),
        out_shape=jax.ShapeDtypeStruct((bm, ncls), jnp.float32),
    )(atoms, bonds_r, edges,
      w1a.astype(jnp.bfloat16), w2a.astype(jnp.bfloat16),
      woa.astype(jnp.bfloat16), st6, ones8, l8, dl48, dl896, wrb, wrbias,
      Wf.astype(jnp.bfloat16), bf2)


# speedup vs baseline: 1.2343x; 1.0295x over previous
import functools

import jax
import jax.numpy as jnp
from jax.experimental import pallas as pl


def _split(x):
    hi = x.astype(jnp.bfloat16)
    r = x - hi.astype(jnp.float32)
    mid = r.astype(jnp.bfloat16)
    lo = (r - mid.astype(jnp.float32)).astype(jnp.bfloat16)
    return hi, mid, lo


def _body(mb, na, nd, nf, nbf, hid, ncls,
          atoms_ref, bonds_ref, edges_ref,
          w1a_ref, w2a_ref, woa_ref,
          st6_ref, ones8_ref, l8_ref, dl48_ref, dl896_ref,
          wrb_ref, wrbias_ref,
          wf_ref, bf_ref, out_ref):
    ndeg = nd + 1
    atoms = atoms_ref[...]
    edges = edges_ref[...]
    bonds = bonds_ref[...]

    vf = (edges != -1).astype(jnp.float32).reshape(mb * na, nd)
    deg8 = jnp.dot(vf, ones8_ref[...],
                   preferred_element_type=jnp.float32)
    deg_col = deg8[:, 0:1]
    nz_col = (deg_col > 0.0).astype(jnp.float32)
    mask_cat = (dl896_ref[...] == deg_col).astype(jnp.float32)

    bflat = bonds.reshape(mb * na, nd * nbf)
    bh = bflat.astype(jnp.bfloat16)
    bl = (bflat - bh.astype(jnp.float32)).astype(jnp.bfloat16)
    b6t = (jnp.dot(bh, st6_ref[...], preferred_element_type=jnp.float32)
           + jnp.dot(bl, st6_ref[...],
                     preferred_element_type=jnp.float32))
    krp = b6t * (dl48_ref[...] == deg_col).astype(jnp.float32)
    dsel = (l8_ref[...] == deg8).astype(jnp.float32)
    q = (jnp.dot(krp, wrb_ref[...], preferred_element_type=jnp.float32)
         + jnp.dot(dsel, wrbias_ref[...],
                   preferred_element_type=jnp.float32))
    q1 = q[:, 0:hid]
    q2 = q[:, hid:2 * hid]
    qo = q[:, 2 * hid:]

    colb = jax.lax.broadcasted_iota(
        jnp.int32, (mb, na, na), 2).astype(jnp.bfloat16)
    eb = edges.astype(jnp.bfloat16)
    onehots = []
    for d in range(nd):
        e_d = eb[:, :, d]
        onehots.append((e_d[:, :, None] == colb).astype(jnp.bfloat16))
    adj = onehots[0]
    for d in range(1, nd):
        adj = adj + onehots[d]
    oh_cat = jnp.concatenate(onehots, axis=1)

    def bmm(a, b):
        return jax.lax.dot_general(
            a, b, (((2,), (1,)), ((0,), (0,))),
            preferred_element_type=jnp.float32)

    def gather2(oh, x):
        hi, mid, lo = _split(x)
        return bmm(oh, hi) + bmm(oh, mid) + bmm(oh, lo)

    def conv(x, wa_ref, qq, in_dim):
        summed = x + gather2(adj, x)
        s = summed.reshape(mb * na, in_dim)
        y = jnp.dot(s.astype(jnp.bfloat16), wa_ref[...],
                    preferred_element_type=jnp.float32) * mask_cat
        out = qq + y[:, 0:hid]
        for d in range(1, ndeg):
            out = out + y[:, d * hid:(d + 1) * hid]
        return jnp.maximum(out, 0.0).reshape(mb, na, hid)

    def pool(x):
        g_all = gather2(oh_cat, x)
        p = x
        for d in range(nd):
            p = jnp.maximum(p, g_all[:, d * na:(d + 1) * na, :])
        return p * nz_col.reshape(mb, na, 1)

    x1 = conv(atoms, w1a_ref, q1, nf)
    p1 = pool(x1)
    x2 = conv(p1, w2a_ref, q2, hid)
    p2 = pool(x2)

    f = jnp.tanh(jnp.dot(p2.reshape(mb * na, hid).astype(jnp.bfloat16),
                         woa_ref[...],
                         preferred_element_type=jnp.float32)
                 + qo)
    f = f * nz_col
    fp = f.reshape(mb, na, hid).sum(axis=1)
    logits = (jnp.dot(fp.astype(jnp.bfloat16), wf_ref[...],
                      preferred_element_type=jnp.float32)
              + bf_ref[...])
    out_ref[...] = jax.nn.sigmoid(logits)


@jax.jit
def kernel(atoms, bonds, edges, W1, b1, W2, b2, Wo, bo, Wf, bf):
    bm, na, nf = atoms.shape
    nd = edges.shape[-1]
    nbf = bonds.shape[-1]
    ndeg, in1, hid = W1.shape
    ncls = Wf.shape[-1]
    mb = 32
    grid = (bm // mb,)

    bonds_r = bonds.reshape(bm, na, nd * nbf)
    w1c = W1.transpose(1, 0, 2).reshape(in1, ndeg * hid)
    w1a, w1b = w1c[:nf], w1c[nf:]
    b1c = b1.reshape(1, ndeg * hid)
    in2 = W2.shape[1]
    w2c = W2.transpose(1, 0, 2).reshape(in2, ndeg * hid)
    w2a, w2b = w2c[:hid], w2c[hid:]
    b2c = b2.reshape(1, ndeg * hid)
    woa, wob = Wo[:hid], Wo[hid:]
    bf2 = bf.reshape(1, ncls)
    kb, kbp = ndeg * nbf, 48
    st6 = jnp.tile(jnp.eye(nbf, dtype=jnp.float32), (nd, 8))[:, :kbp]
    ones8 = jnp.ones((nd, 8), jnp.float32)
    l8 = jnp.arange(8, dtype=jnp.float32).reshape(1, 8)
    dl48 = (jnp.arange(kbp, dtype=jnp.int32) // nbf).astype(jnp.float32).reshape(1, kbp)
    dl896 = (jnp.arange(ndeg * hid, dtype=jnp.int32) // hid).astype(
        jnp.float32).reshape(1, ndeg * hid)
    w1b_r = w1b.reshape(nbf, ndeg, hid).transpose(1, 0, 2).reshape(kb, hid)
    w2b_r = w2b.reshape(nbf, ndeg, hid).transpose(1, 0, 2).reshape(kb, hid)
    wob_r = jnp.tile(wob, (ndeg, 1))
    wrb = jnp.concatenate(
        [jnp.concatenate([w1b_r, w2b_r, wob_r], axis=1),
         jnp.zeros((kbp - kb, 3 * hid), jnp.float32)], axis=0)
    wrbias = jnp.concatenate(
        [jnp.concatenate([b1, b2, jnp.tile(bo[None], (ndeg, 1))], axis=1),
         jnp.zeros((1, 3 * hid), jnp.float32)], axis=0)

    const = lambda *shape: pl.BlockSpec(shape, lambda i: (0,) * len(shape))
    return pl.pallas_call(
        functools.partial(_body, mb, na, nd, nf, nbf, hid, ncls),
        grid=grid,
        in_specs=[
            pl.BlockSpec((mb, na, nf), lambda i: (i, 0, 0)),
            pl.BlockSpec((mb, na, nd * nbf), lambda i: (i, 0, 0)),
            pl.BlockSpec((mb, na, nd), lambda i: (i, 0, 0)),
            const(nf, ndeg * hid),
            const(hid, ndeg * hid),
            const(hid, hid),
            const(nd * nbf, 48),
            const(nd, 8),
            const(1, 8),
            const(1, 48),
            const(1, ndeg * hid),
            const(48, 3 * hid),
            const(8, 3 * hid),
            const(hid, ncls),
            const(1, ncls),
        ],
        out_specs=pl.BlockSpec((mb, ncls), lambda i: (i, 0)),
        out_shape=jax.ShapeDtypeStruct((bm, ncls), jnp.float32),
    )(atoms, bonds_r, edges,
      w1a.astype(jnp.bfloat16), w2a.astype(jnp.bfloat16),
      woa.astype(jnp.bfloat16), st6, ones8, l8, dl48, dl896, wrb, wrbias,
      Wf.astype(jnp.bfloat16), bf2)


# speedup vs baseline: 1.3170x; 1.0670x over previous
import functools

import jax
import jax.numpy as jnp
from jax.experimental import pallas as pl


def _split(x):
    hi = x.astype(jnp.bfloat16)
    r = x - hi.astype(jnp.float32)
    mid = r.astype(jnp.bfloat16)
    lo = (r - mid.astype(jnp.float32)).astype(jnp.bfloat16)
    return hi, mid, lo


def _body(mb, na, nd, nf, nbf, hid, ncls,
          atoms_ref, bonds_ref, edges_ref,
          w1a_ref, w2a_ref, woa_ref,
          st6_ref, ones8_ref, l8_ref, dl48_ref, dl896_ref,
          wrb_ref, wrbias_ref,
          wf_ref, bf_ref, out_ref):
    ndeg = nd + 1
    atoms = atoms_ref[...]
    edges = edges_ref[...]
    bonds = bonds_ref[...]

    vf = (edges != -1).astype(jnp.float32).reshape(mb * na, nd)
    deg8 = jnp.dot(vf, ones8_ref[...],
                   preferred_element_type=jnp.float32)
    deg_col = deg8[:, 0:1]
    nz_col = (deg_col > 0.0).astype(jnp.float32)
    mask_cat = (dl896_ref[...] == deg_col).astype(jnp.float32)

    bflat = bonds.reshape(mb * na, nd * nbf)
    bh = bflat.astype(jnp.bfloat16)
    bl = (bflat - bh.astype(jnp.float32)).astype(jnp.bfloat16)
    b6t = (jnp.dot(bh, st6_ref[...], preferred_element_type=jnp.float32)
           + jnp.dot(bl, st6_ref[...],
                     preferred_element_type=jnp.float32))
    krp = b6t * (dl48_ref[...] == deg_col).astype(jnp.float32)
    dsel = (l8_ref[...] == deg8).astype(jnp.float32)
    q = (jnp.dot(krp, wrb_ref[...], preferred_element_type=jnp.float32)
         + jnp.dot(dsel, wrbias_ref[...],
                   preferred_element_type=jnp.float32))
    q1 = q[:, 0:hid]
    q2 = q[:, hid:2 * hid]
    qo = q[:, 2 * hid:]

    colb = jax.lax.broadcasted_iota(
        jnp.int32, (mb, na, na), 2).astype(jnp.bfloat16)
    eb = edges.astype(jnp.bfloat16)
    onehots = []
    for d in range(nd):
        e_d = eb[:, :, d]
        onehots.append((e_d[:, :, None] == colb).astype(jnp.bfloat16))
    adj = onehots[0]
    for d in range(1, nd):
        adj = adj + onehots[d]
    oh_cat = jnp.concatenate(onehots, axis=1)

    def bmm(a, b):
        return jax.lax.dot_general(
            a, b, (((2,), (1,)), ((0,), (0,))),
            preferred_element_type=jnp.float32)

    def gather2(oh, x):
        hi, mid, lo = _split(x)
        return bmm(oh, hi) + bmm(oh, mid) + bmm(oh, lo)

    def gathersum2(oh, x):
        hi = x.astype(jnp.bfloat16)
        lo = (x - hi.astype(jnp.float32)).astype(jnp.bfloat16)
        return bmm(oh, hi) + bmm(oh, lo)

    def conv(x, wa_ref, qq, in_dim):
        summed = x + gathersum2(adj, x)
        s = summed.reshape(mb * na, in_dim)
        y = jnp.dot(s.astype(jnp.bfloat16), wa_ref[...],
                    preferred_element_type=jnp.float32) * mask_cat
        out = qq + y[:, 0:hid]
        for d in range(1, ndeg):
            out = out + y[:, d * hid:(d + 1) * hid]
        return jnp.maximum(out, 0.0).reshape(mb, na, hid)

    def pool(x):
        g_all = gather2(oh_cat, x)
        p = x
        for d in range(nd):
            p = jnp.maximum(p, g_all[:, d * na:(d + 1) * na, :])
        return p * nz_col.reshape(mb, na, 1)

    x1 = conv(atoms, w1a_ref, q1, nf)
    p1 = pool(x1)
    x2 = conv(p1, w2a_ref, q2, hid)
    p2 = pool(x2)

    f = jnp.tanh(jnp.dot(p2.reshape(mb * na, hid).astype(jnp.bfloat16),
                         woa_ref[...],
                         preferred_element_type=jnp.float32)
                 + qo)
    f = f * nz_col
    fp = f.reshape(mb, na, hid).sum(axis=1)
    logits = (jnp.dot(fp.astype(jnp.bfloat16), wf_ref[...],
                      preferred_element_type=jnp.float32)
              + bf_ref[...])
    out_ref[...] = jax.nn.sigmoid(logits)


@jax.jit
def kernel(atoms, bonds, edges, W1, b1, W2, b2, Wo, bo, Wf, bf):
    bm, na, nf = atoms.shape
    nd = edges.shape[-1]
    nbf = bonds.shape[-1]
    ndeg, in1, hid = W1.shape
    ncls = Wf.shape[-1]
    mb = 32
    grid = (bm // mb,)

    bonds_r = bonds.reshape(bm, na, nd * nbf)
    w1c = W1.transpose(1, 0, 2).reshape(in1, ndeg * hid)
    w1a, w1b = w1c[:nf], w1c[nf:]
    b1c = b1.reshape(1, ndeg * hid)
    in2 = W2.shape[1]
    w2c = W2.transpose(1, 0, 2).reshape(in2, ndeg * hid)
    w2a, w2b = w2c[:hid], w2c[hid:]
    b2c = b2.reshape(1, ndeg * hid)
    woa, wob = Wo[:hid], Wo[hid:]
    bf2 = bf.reshape(1, ncls)
    kb, kbp = ndeg * nbf, 48
    st6 = jnp.tile(jnp.eye(nbf, dtype=jnp.float32), (nd, 8))[:, :kbp]
    ones8 = jnp.ones((nd, 8), jnp.float32)
    l8 = jnp.arange(8, dtype=jnp.float32).reshape(1, 8)
    dl48 = (jnp.arange(kbp, dtype=jnp.int32) // nbf).astype(jnp.float32).reshape(1, kbp)
    dl896 = (jnp.arange(ndeg * hid, dtype=jnp.int32) // hid).astype(
        jnp.float32).reshape(1, ndeg * hid)
    w1b_r = w1b.reshape(nbf, ndeg, hid).transpose(1, 0, 2).reshape(kb, hid)
    w2b_r = w2b.reshape(nbf, ndeg, hid).transpose(1, 0, 2).reshape(kb, hid)
    wob_r = jnp.tile(wob, (ndeg, 1))
    wrb = jnp.concatenate(
        [jnp.concatenate([w1b_r, w2b_r, wob_r], axis=1),
         jnp.zeros((kbp - kb, 3 * hid), jnp.float32)], axis=0)
    wrbias = jnp.concatenate(
        [jnp.concatenate([b1, b2, jnp.tile(bo[None], (ndeg, 1))], axis=1),
         jnp.zeros((1, 3 * hid), jnp.float32)], axis=0)

    const = lambda *shape: pl.BlockSpec(shape, lambda i: (0,) * len(shape))
    return pl.pallas_call(
        functools.partial(_body, mb, na, nd, nf, nbf, hid, ncls),
        grid=grid,
        in_specs=[
            pl.BlockSpec((mb, na, nf), lambda i: (i, 0, 0)),
            pl.BlockSpec((mb, na, nd * nbf), lambda i: (i, 0, 0)),
            pl.BlockSpec((mb, na, nd), lambda i: (i, 0, 0)),
            const(nf, ndeg * hid),
            const(hid, ndeg * hid),
            const(hid, hid),
            const(nd * nbf, 48),
            const(nd, 8),
            const(1, 8),
            const(1, 48),
            const(1, ndeg * hid),
            const(48, 3 * hid),
            const(8, 3 * hid),
            const(hid, ncls),
            const(1, ncls),
        ],
        out_specs=pl.BlockSpec((mb, ncls), lambda i: (i, 0)),
        out_shape=jax.ShapeDtypeStruct((bm, ncls), jnp.float32),
    )(atoms, bonds_r, edges,
      w1a.astype(jnp.bfloat16), w2a.astype(jnp.bfloat16),
      woa.astype(jnp.bfloat16), st6, ones8, l8, dl48, dl896, wrb, wrbias,
      Wf.astype(jnp.bfloat16), bf2)


# speedup vs baseline: 1.3186x; 1.0013x over previous
import functools

import jax
import jax.numpy as jnp
from jax.experimental import pallas as pl


def _split(x):
    hi = x.astype(jnp.bfloat16)
    r = x - hi.astype(jnp.float32)
    mid = r.astype(jnp.bfloat16)
    lo = (r - mid.astype(jnp.float32)).astype(jnp.bfloat16)
    return hi, mid, lo


def _body(mb, na, nd, nf, nbf, hid, ncls,
          atoms_ref, bonds_ref, edges_ref,
          w1a_ref, w2a_ref, woa_ref,
          st6_ref, ones8_ref, l8_ref, dl48_ref, dl896_ref,
          wrb_ref, wrbias_ref,
          wf_ref, bf_ref, out_ref):
    ndeg = nd + 1
    atoms = atoms_ref[...]
    edges = edges_ref[...]
    bonds = bonds_ref[...]

    vf = (edges != -1).astype(jnp.float32).reshape(mb * na, nd)
    deg8 = jnp.dot(vf, ones8_ref[...],
                   preferred_element_type=jnp.float32)
    deg_col = deg8[:, 0:1]
    nz_col = (deg_col > 0.0).astype(jnp.float32)
    mask_cat = (dl896_ref[...] == deg_col).astype(jnp.float32)

    bflat = bonds.reshape(mb * na, nd * nbf)
    bh = bflat.astype(jnp.bfloat16)
    bl = (bflat - bh.astype(jnp.float32)).astype(jnp.bfloat16)
    b6t = (jnp.dot(bh, st6_ref[...], preferred_element_type=jnp.float32)
           + jnp.dot(bl, st6_ref[...],
                     preferred_element_type=jnp.float32))
    krp = b6t * (dl48_ref[...] == deg_col).astype(jnp.float32)
    dsel = (l8_ref[...] == deg8).astype(jnp.float32)
    q = (jnp.dot(krp, wrb_ref[...], preferred_element_type=jnp.float32)
         + jnp.dot(dsel, wrbias_ref[...],
                   preferred_element_type=jnp.float32))
    q1 = q[:, 0:hid]
    q2 = q[:, hid:2 * hid]
    qo = q[:, 2 * hid:]

    colb = jax.lax.broadcasted_iota(
        jnp.int32, (mb, na, na), 2).astype(jnp.bfloat16)
    eb = edges.astype(jnp.bfloat16)
    onehots = []
    for d in range(nd):
        e_d = eb[:, :, d]
        onehots.append((e_d[:, :, None] == colb).astype(jnp.bfloat16))
    adj = onehots[0]
    for d in range(1, nd):
        adj = adj + onehots[d]
    oh_cat = jnp.concatenate(onehots, axis=1)

    def bmm(a, b):
        return jax.lax.dot_general(
            a, b, (((2,), (1,)), ((0,), (0,))),
            preferred_element_type=jnp.float32)

    def gather2(oh, x):
        hi, mid, lo = _split(x)
        return bmm(oh, hi) + bmm(oh, mid) + bmm(oh, lo)

    def gathersum2(oh, x):
        hi = x.astype(jnp.bfloat16)
        lo = (x - hi.astype(jnp.float32)).astype(jnp.bfloat16)
        return bmm(oh, hi) + bmm(oh, lo)

    def conv(x, wa_ref, qq, in_dim):
        summed = x + gathersum2(adj, x)
        s = summed.reshape(mb * na, in_dim)
        y = jnp.dot(s.astype(jnp.bfloat16), wa_ref[...],
                    preferred_element_type=jnp.float32) * mask_cat
        out = qq + y[:, 0:hid]
        for d in range(1, ndeg):
            out = out + y[:, d * hid:(d + 1) * hid]
        return jnp.maximum(out, 0.0).reshape(mb, na, hid)

    def pool(x):
        g_all = gather2(oh_cat, x)
        p = x
        for d in range(nd):
            p = jnp.maximum(p, g_all[:, d * na:(d + 1) * na, :])
        return p * nz_col.reshape(mb, na, 1)

    x1 = conv(atoms, w1a_ref, q1, nf)
    p1 = pool(x1)
    x2 = conv(p1, w2a_ref, q2, hid)
    p2 = pool(x2)

    f = jnp.tanh(jnp.dot(p2.reshape(mb * na, hid).astype(jnp.bfloat16),
                         woa_ref[...],
                         preferred_element_type=jnp.float32)
                 + qo)
    f = f * nz_col
    fp = f.reshape(mb, na, hid).sum(axis=1)
    logits = (jnp.dot(fp.astype(jnp.bfloat16), wf_ref[...],
                      preferred_element_type=jnp.float32)
              + bf_ref[...])
    out_ref[...] = jax.nn.sigmoid(logits)


@jax.jit
def kernel(atoms, bonds, edges, W1, b1, W2, b2, Wo, bo, Wf, bf):
    bm, na, nf = atoms.shape
    nd = edges.shape[-1]
    nbf = bonds.shape[-1]
    ndeg, in1, hid = W1.shape
    ncls = Wf.shape[-1]
    mb = 32
    grid = (bm // mb,)

    bonds_r = bonds.reshape(bm, na, nd * nbf)
    w1c = W1.transpose(1, 0, 2).reshape(in1, ndeg * hid)
    w1a, w1b = w1c[:nf], w1c[nf:]
    in2 = W2.shape[1]
    w2c = W2.transpose(1, 0, 2).reshape(in2, ndeg * hid)
    w2a, w2b = w2c[:hid], w2c[hid:]
    woa, wob = Wo[:hid], Wo[hid:]
    bf2 = bf.reshape(1, ncls)
    kb, kbp = ndeg * nbf, 48
    st6 = jnp.tile(jnp.eye(nbf, dtype=jnp.float32), (nd, 8))[:, :kbp]
    ones8 = jnp.ones((nd, 8), jnp.float32)
    l8 = jnp.arange(8, dtype=jnp.float32).reshape(1, 8)
    dl48 = (jnp.arange(kbp, dtype=jnp.int32) // nbf).astype(jnp.float32).reshape(1, kbp)
    dl896 = (jnp.arange(ndeg * hid, dtype=jnp.int32) // hid).astype(
        jnp.float32).reshape(1, ndeg * hid)
    w1b_r = w1b.reshape(nbf, ndeg, hid).transpose(1, 0, 2).reshape(kb, hid)
    w2b_r = w2b.reshape(nbf, ndeg, hid).transpose(1, 0, 2).reshape(kb, hid)
    wob_r = jnp.tile(wob, (ndeg, 1))
    wrb = jnp.concatenate(
        [jnp.concatenate([w1b_r, w2b_r, wob_r], axis=1),
         jnp.zeros((kbp - kb, 3 * hid), jnp.float32)], axis=0)
    wrbias = jnp.concatenate(
        [jnp.concatenate([b1, b2, jnp.tile(bo[None], (ndeg, 1))], axis=1),
         jnp.zeros((1, 3 * hid), jnp.float32)], axis=0)

    const = lambda *shape: pl.BlockSpec(shape, lambda i: (0,) * len(shape))
    return pl.pallas_call(
        functools.partial(_body, mb, na, nd, nf, nbf, hid, ncls),
        grid=grid,
        in_specs=[
            pl.BlockSpec((mb, na, nf), lambda i: (i, 0, 0)),
            pl.BlockSpec((mb, na, nd * nbf), lambda i: (i, 0, 0)),
            pl.BlockSpec((mb, na, nd), lambda i: (i, 0, 0)),
            const(nf, ndeg * hid),
            const(hid, ndeg * hid),
            const(hid, hid),
            const(nd * nbf, 48),
            const(nd, 8),
            const(1, 8),
            const(1, 48),
            const(1, ndeg * hid),
            const(48, 3 * hid),
            const(8, 3 * hid),
            const(hid, ncls),
            const(1, ncls),
        ],
        out_specs=pl.BlockSpec((mb, ncls), lambda i: (i, 0)),
        out_shape=jax.ShapeDtypeStruct((bm, ncls), jnp.float32),
    )(atoms, bonds_r, edges,
      w1a.astype(jnp.bfloat16), w2a.astype(jnp.bfloat16),
      woa.astype(jnp.bfloat16), st6, ones8, l8, dl48, dl896, wrb, wrbias,
      Wf.astype(jnp.bfloat16), bf2)
